# Initial kernel scaffold; baseline (speedup 1.0000x reference)
import jax
import jax.numpy as jnp
from jax.experimental import pallas as pl


def kernel(x, indices):
    raise NotImplementedError("write your pallas kernel here")


# speedup vs baseline: 9.5945x; 9.5945x over previous
import functools

import jax
import jax.numpy as jnp
from jax import lax
from jax.experimental import pallas as pl
from jax.experimental.pallas import tpu as pltpu
from jax.experimental.pallas import tpu_sc as plsc

N_NODES = 100000
D = 128
M = 200000
K = 3

_LANES = 16
_B = 128


def _build(nc: int, ns: int):
    nw = nc * ns
    mpw = M // nw
    n_full = mpw // _B
    tail = mpw - n_full * _B
    n_chunk = n_full + (1 if tail else 0)

    mesh = plsc.VectorSubcoreMesh(core_axis_name="c", subcore_axis_name="s")

    @functools.partial(
        pl.kernel,
        mesh=mesh,
        compiler_params=pltpu.CompilerParams(use_tc_tiling_on_sc=False),
        out_type=jax.ShapeDtypeStruct((M, D), jnp.float32),
        scratch_types=[
            pltpu.VMEM((K, n_chunk, _B), jnp.int32),
            pltpu.VMEM((_B, D), jnp.float32),
            pltpu.VMEM((_B, D), jnp.float32),
            pltpu.VMEM((_B, D), jnp.float32),
            pltpu.SemaphoreType.DMA,
        ],
    )
    def sc_kernel(x_hbm, idx_hbm, out_hbm, idx_v, ob, b1, b2, sem):
        wid = lax.axis_index("s") * nc + lax.axis_index("c")
        base = wid * mpw
        pltpu.sync_copy(idx_hbm.at[wid], idx_v)

        def gather_chunk(c):
            d0 = pltpu.async_copy(x_hbm.at[idx_v.at[0, c]], ob, sem)
            d1 = pltpu.async_copy(x_hbm.at[idx_v.at[1, c]], b1, sem)
            d2 = pltpu.async_copy(x_hbm.at[idx_v.at[2, c]], b2, sem)
            d0.wait()
            d1.wait()
            d2.wait()

        def accumulate():
            def row(r, carry):
                for j in range(D // _LANES):
                    sl = pl.ds(j * _LANES, _LANES)
                    plsc.addupdate(ob.at[r, sl], b1[r, sl] + b2[r, sl])
                return carry
            lax.fori_loop(0, _B, row, 0)

        def body(c, carry):
            gather_chunk(c)
            accumulate()
            pltpu.sync_copy(ob, out_hbm.at[pl.ds(base + c * _B, _B)])
            return carry

        lax.fori_loop(0, n_full, body, 0)
        if tail:
            gather_chunk(n_full)
            accumulate()
            pltpu.sync_copy(
                ob.at[pl.ds(0, tail)],
                out_hbm.at[pl.ds(base + n_full * _B, tail)],
            )

    def run(x, indices):
        idx32 = indices.astype(jnp.int32)
        idx_t = idx32.T.reshape(K, nw, mpw).transpose(1, 0, 2)
        pad = n_chunk * _B - mpw
        if pad:
            idx_t = jnp.pad(idx_t, ((0, 0), (0, 0), (0, pad)))
        idx_t = idx_t.reshape(nw, K, n_chunk, _B)
        return sc_kernel(x, idx_t)

    return run


def kernel(x, indices):
    info = plsc.get_sparse_core_info()
    return _build(info.num_cores, info.num_subcores)(x, indices)


# speedup vs baseline: 13.5076x; 1.4078x over previous
import functools

import jax
import jax.numpy as jnp
from jax import lax
from jax.experimental import pallas as pl
from jax.experimental.pallas import tpu as pltpu
from jax.experimental.pallas import tpu_sc as plsc

N_NODES = 100000
D = 128
M = 200000
K = 3

_LANES = 16
_B = 128


def _build(nc: int, ns: int):
    nw = nc * ns
    mpw = M // nw
    n_full = mpw // _B
    tail = mpw - n_full * _B
    n_chunk = n_full + (1 if tail else 0)
    assert tail and n_full >= 2 and n_full % 2 == 0

    mesh = plsc.VectorSubcoreMesh(core_axis_name="c", subcore_axis_name="s")

    @functools.partial(
        pl.kernel,
        mesh=mesh,
        compiler_params=pltpu.CompilerParams(use_tc_tiling_on_sc=False),
        out_type=jax.ShapeDtypeStruct((M, D), jnp.float32),
        scratch_types=[
            pltpu.VMEM((K, n_chunk, _B), jnp.int32),
            pltpu.VMEM((_B, D), jnp.float32),
            pltpu.VMEM((_B, D), jnp.float32),
            pltpu.VMEM((_B, D), jnp.float32),
            pltpu.VMEM((_B, D), jnp.float32),
            pltpu.VMEM((_B, D), jnp.float32),
            pltpu.VMEM((_B, D), jnp.float32),
            pltpu.SemaphoreType.DMA,
            pltpu.SemaphoreType.DMA,
            pltpu.SemaphoreType.DMA,
            pltpu.SemaphoreType.DMA,
        ],
    )
    def sc_kernel(x_hbm, idx_hbm, out_hbm, idx_v,
                  ob0, b10, b20, ob1, b11, b21,
                  semg0, sems0, semg1, sems1):
        wid = lax.axis_index("s") * nc + lax.axis_index("c")
        base = wid * mpw
        pltpu.sync_copy(idx_hbm.at[wid], idx_v)

        set0 = (ob0, b10, b20, semg0, sems0)
        set1 = (ob1, b11, b21, semg1, sems1)

        def fire_gathers(c, st):
            ob, b1, b2, semg, _ = st
            pltpu.async_copy(x_hbm.at[idx_v.at[0, c]], ob, semg)
            pltpu.async_copy(x_hbm.at[idx_v.at[1, c]], b1, semg)
            pltpu.async_copy(x_hbm.at[idx_v.at[2, c]], b2, semg)

        def wait_gathers(c, st):
            ob, b1, b2, semg, _ = st
            pltpu.make_async_copy(x_hbm.at[idx_v.at[0, c]], ob, semg).wait()
            pltpu.make_async_copy(x_hbm.at[idx_v.at[1, c]], b1, semg).wait()
            pltpu.make_async_copy(x_hbm.at[idx_v.at[2, c]], b2, semg).wait()

        def fire_store(c, st):
            pltpu.async_copy(st[0], out_hbm.at[pl.ds(base + c * _B, _B)], st[4])

        def wait_store(c, st):
            pltpu.make_async_copy(
                st[0], out_hbm.at[pl.ds(base + c * _B, _B)], st[4]).wait()

        def accumulate(st):
            ob, b1, b2 = st[0], st[1], st[2]

            def row(r, carry):
                for j in range(D // _LANES):
                    sl = pl.ds(j * _LANES, _LANES)
                    plsc.addupdate(ob.at[r, sl], b1[r, sl] + b2[r, sl])
                return carry

            lax.fori_loop(0, _B, row, 0)

        def step(c, cur, nxt):
            wait_store(c - 1, nxt)
            fire_gathers(c + 1, nxt)
            wait_gathers(c, cur)
            accumulate(cur)
            fire_store(c, cur)

        fire_gathers(0, set0)
        fire_gathers(1, set1)
        wait_gathers(0, set0)
        accumulate(set0)
        fire_store(0, set0)
        step(1, set1, set0)

        def pair(j, carry):
            step(2 * j + 2, set0, set1)
            step(2 * j + 3, set1, set0)
            return carry

        lax.fori_loop(0, (n_full - 2) // 2, pair, 0)

        wait_store(n_full - 1, set1)
        wait_gathers(n_full, set0)
        accumulate(set0)
        pltpu.sync_copy(
            ob0.at[pl.ds(0, tail)],
            out_hbm.at[pl.ds(base + n_full * _B, tail)],
        )

    def run(x, indices):
        idx32 = indices.astype(jnp.int32)
        idx_t = idx32.T.reshape(K, nw, mpw).transpose(1, 0, 2)
        pad = n_chunk * _B - mpw
        if pad:
            idx_t = jnp.pad(idx_t, ((0, 0), (0, 0), (0, pad)))
        idx_t = idx_t.reshape(nw, K, n_chunk, _B)
        return sc_kernel(x, idx_t)

    return run


def kernel(x, indices):
    info = plsc.get_sparse_core_info()
    return _build(info.num_cores, info.num_subcores)(x, indices)
